# Initial kernel scaffold; baseline (speedup 1.0000x reference)
import functools

import jax
import jax.numpy as jnp
from jax import lax
from jax.experimental import pallas as pl
from jax.experimental.pallas import tpu as pltpu
from jax.experimental.pallas import tpu_sc as plsc

B = 4096
U = 4096
I = 8192
D = 64

NC = 2
NS = 16
NW = NC * NS
L = 16
BPW = B // NW
NG = BPW // L

_mesh = plsc.VectorSubcoreMesh(core_axis_name="c", subcore_axis_name="s")


@functools.partial(
    pl.kernel,
    mesh=_mesh,
    out_type=(
        jax.ShapeDtypeStruct((B,), jnp.float32),
        jax.ShapeDtypeStruct((B,), jnp.float32),
    ),
    scratch_types=dict(
        idx_u=pltpu.VMEM((BPW,), jnp.int32),
        idx_i=pltpu.VMEM((BPW,), jnp.int32),
        idx_j=pltpu.VMEM((BPW,), jnp.int32),
        idx_il=pltpu.VMEM((BPW,), jnp.int32),
        idx_jl=pltpu.VMEM((BPW,), jnp.int32),
        u_rows=pltpu.VMEM((BPW, D), jnp.float32),
        ei_rows=pltpu.VMEM((BPW, D), jnp.float32),
        ej_rows=pltpu.VMEM((BPW, D), jnp.float32),
        lin_u=pltpu.VMEM((BPW,), jnp.float32),
        lin_iv=pltpu.VMEM((BPW,), jnp.float32),
        lin_jv=pltpu.VMEM((BPW,), jnp.float32),
        out_i_v=pltpu.VMEM((BPW,), jnp.float32),
        out_j_v=pltpu.VMEM((BPW,), jnp.float32),
        sem=pltpu.SemaphoreType.DMA,
    ),
)
def _fm_kernel(
    user_hbm, item_i_hbm, item_j_hbm, lin_hbm, eu_hbm, eit_hbm,
    out_i_hbm, out_j_hbm,
    *, idx_u, idx_i, idx_j, idx_il, idx_jl,
    u_rows, ei_rows, ej_rows, lin_u, lin_iv, lin_jv,
    out_i_v, out_j_v, sem,
):
    wid = lax.axis_index("s") * NC + lax.axis_index("c")
    base = wid * BPW

    pltpu.sync_copy(user_hbm.at[pl.ds(base, BPW)], idx_u)
    pltpu.sync_copy(item_i_hbm.at[pl.ds(base, BPW)], idx_i)
    pltpu.sync_copy(item_j_hbm.at[pl.ds(base, BPW)], idx_j)

    cps = [
        pltpu.async_copy(eu_hbm.at[idx_u], u_rows, sem),
        pltpu.async_copy(eit_hbm.at[idx_i], ei_rows, sem),
        pltpu.async_copy(eit_hbm.at[idx_j], ej_rows, sem),
    ]

    off = jnp.full((L,), U, jnp.int32)
    for c in range(BPW // L):
        sl = pl.ds(c * L, L)
        idx_il[sl] = idx_i[sl] + off
        idx_jl[sl] = idx_j[sl] + off
    cps.append(pltpu.async_copy(lin_hbm.at[idx_u], lin_u, sem))
    cps.append(pltpu.async_copy(lin_hbm.at[idx_il], lin_iv, sem))
    cps.append(pltpu.async_copy(lin_hbm.at[idx_jl], lin_jv, sem))
    for cp in cps:
        cp.wait()

    iota = lax.iota(jnp.int32, L)
    zeros = jnp.zeros((L,), jnp.float32)
    for g in range(NG):
        rows = iota + jnp.full((L,), g * L, jnp.int32)

        def dstep(d, carry, rows=rows):
            acc_i, acc_j = carry
            dcol = jnp.full((L,), d, jnp.int32)
            ucol = plsc.load_gather(u_rows, [rows, dcol])
            eicol = plsc.load_gather(ei_rows, [rows, dcol])
            ejcol = plsc.load_gather(ej_rows, [rows, dcol])
            return acc_i + ucol * eicol, acc_j + ucol * ejcol

        acc_i, acc_j = lax.fori_loop(0, D, dstep, (zeros, zeros))
        sl = pl.ds(g * L, L)
        out_i_v[sl] = acc_i + 0.3 * (lin_u[sl] + lin_iv[sl])
        out_j_v[sl] = acc_j + 0.3 * (lin_u[sl] + lin_jv[sl])

    pltpu.sync_copy(out_i_v, out_i_hbm.at[pl.ds(base, BPW)])
    pltpu.sync_copy(out_j_v, out_j_hbm.at[pl.ds(base, BPW)])


def kernel(user, item_i, item_j, linear_w, embed_user_w, embed_item_w):
    user = user.astype(jnp.int32)
    item_i = item_i.astype(jnp.int32)
    item_j = item_j.astype(jnp.int32)
    lin_flat = linear_w.reshape(-1)
    return _fm_kernel(user, item_i, item_j, lin_flat, embed_user_w,
                      embed_item_w)


# speedup vs baseline: 9.1116x; 9.1116x over previous
import functools

import jax
import jax.numpy as jnp
from jax import lax
from jax.experimental import pallas as pl
from jax.experimental.pallas import tpu as pltpu
from jax.experimental.pallas import tpu_sc as plsc

B = 4096
U = 4096
I = 8192
D = 64

NC = 2
NS = 16
NW = NC * NS
L = 16
BPW = B // NW
NG = BPW // L

_mesh = plsc.VectorSubcoreMesh(core_axis_name="c", subcore_axis_name="s")


@functools.partial(
    pl.kernel,
    mesh=_mesh,
    out_type=(
        jax.ShapeDtypeStruct((B,), jnp.float32),
        jax.ShapeDtypeStruct((B,), jnp.float32),
    ),
    scratch_types=dict(
        idx_u=pltpu.VMEM((BPW,), jnp.int32),
        idx_i=pltpu.VMEM((BPW,), jnp.int32),
        idx_j=pltpu.VMEM((BPW,), jnp.int32),
        idx_il=pltpu.VMEM((BPW,), jnp.int32),
        idx_jl=pltpu.VMEM((BPW,), jnp.int32),
        u_rows=pltpu.VMEM((BPW, D), jnp.float32),
        ei_rows=pltpu.VMEM((BPW, D), jnp.float32),
        ej_rows=pltpu.VMEM((BPW, D), jnp.float32),
        lin_u=pltpu.VMEM((BPW,), jnp.float32),
        lin_iv=pltpu.VMEM((BPW,), jnp.float32),
        lin_jv=pltpu.VMEM((BPW,), jnp.float32),
        out_i_v=pltpu.VMEM((BPW,), jnp.float32),
        out_j_v=pltpu.VMEM((BPW,), jnp.float32),
        sem=pltpu.SemaphoreType.DMA,
    ),
    compiler_params=pltpu.CompilerParams(
        needs_layout_passes=False, use_tc_tiling_on_sc=False),
)
def _fm_kernel(
    user_hbm, item_i_hbm, item_j_hbm, lin_hbm, eu_hbm, eit_hbm,
    out_i_hbm, out_j_hbm,
    *, idx_u, idx_i, idx_j, idx_il, idx_jl,
    u_rows, ei_rows, ej_rows, lin_u, lin_iv, lin_jv,
    out_i_v, out_j_v, sem,
):
    wid = lax.axis_index("s") * NC + lax.axis_index("c")
    base = wid * BPW

    pltpu.sync_copy(user_hbm.at[pl.ds(base, BPW)], idx_u)
    pltpu.sync_copy(item_i_hbm.at[pl.ds(base, BPW)], idx_i)
    pltpu.sync_copy(item_j_hbm.at[pl.ds(base, BPW)], idx_j)

    cps = [
        pltpu.async_copy(eu_hbm.at[idx_u], u_rows, sem),
        pltpu.async_copy(eit_hbm.at[idx_i], ei_rows, sem),
        pltpu.async_copy(eit_hbm.at[idx_j], ej_rows, sem),
    ]

    off = jnp.full((L,), U, jnp.int32)
    for c in range(BPW // L):
        sl = pl.ds(c * L, L)
        idx_il[sl] = idx_i[sl] + off
        idx_jl[sl] = idx_j[sl] + off
    cps.append(pltpu.async_copy(lin_hbm.at[idx_u], lin_u, sem))
    cps.append(pltpu.async_copy(lin_hbm.at[idx_il], lin_iv, sem))
    cps.append(pltpu.async_copy(lin_hbm.at[idx_jl], lin_jv, sem))
    for cp in cps:
        cp.wait()

    iota = lax.iota(jnp.int32, L)
    zeros = jnp.zeros((L,), jnp.float32)
    for g in range(NG):
        rows = iota + jnp.full((L,), g * L, jnp.int32)

        def dstep(d, carry, rows=rows):
            acc_i, acc_j = carry
            dcol = jnp.full((L,), d, jnp.int32)
            ucol = plsc.load_gather(u_rows, [rows, dcol])
            eicol = plsc.load_gather(ei_rows, [rows, dcol])
            ejcol = plsc.load_gather(ej_rows, [rows, dcol])
            return acc_i + ucol * eicol, acc_j + ucol * ejcol

        acc_i, acc_j = lax.fori_loop(0, D, dstep, (zeros, zeros))
        sl = pl.ds(g * L, L)
        out_i_v[sl] = acc_i + 0.3 * (lin_u[sl] + lin_iv[sl])
        out_j_v[sl] = acc_j + 0.3 * (lin_u[sl] + lin_jv[sl])

    pltpu.sync_copy(out_i_v, out_i_hbm.at[pl.ds(base, BPW)])
    pltpu.sync_copy(out_j_v, out_j_hbm.at[pl.ds(base, BPW)])


def kernel(user, item_i, item_j, linear_w, embed_user_w, embed_item_w):
    user = user.astype(jnp.int32)
    item_i = item_i.astype(jnp.int32)
    item_j = item_j.astype(jnp.int32)
    lin_flat = linear_w.reshape(-1)
    return _fm_kernel(user, item_i, item_j, lin_flat, embed_user_w,
                      embed_item_w)
